# baseline (speedup 1.0000x reference)
import functools

import jax
import jax.numpy as jnp
from jax import lax
from jax.experimental import pallas as pl
from jax.experimental.pallas import tpu as pltpu
from jax.experimental.pallas import tpu_sc as plsc

VOCAB = 1000000
EMBED = 64
MAX_SEQ = 200
BATCH = 4096

NUM_CORES = 2
NUM_SUBCORES = 16
NUM_WORKERS = NUM_CORES * NUM_SUBCORES
SEQ_PER_WORKER = BATCH // NUM_WORKERS
LANES = 16
VREGS_PER_ROW = EMBED // LANES

_mesh = plsc.VectorSubcoreMesh(core_axis_name="c", subcore_axis_name="s")


@functools.partial(
    pl.kernel,
    mesh=_mesh,
    compiler_params=pltpu.CompilerParams(use_tc_tiling_on_sc=False),
    out_type=jax.ShapeDtypeStruct((BATCH * MAX_SEQ, EMBED), jnp.float32),
    scratch_types=[
        pltpu.VMEM((MAX_SEQ, EMBED), jnp.float32),
        pltpu.VMEM((MAX_SEQ,), jnp.int32),
        pltpu.VMEM((MAX_SEQ, EMBED), jnp.float32),
        pltpu.SemaphoreType.DMA,
    ],
)
def _embed(idx_hbm, tok_hbm, pos_hbm, out_hbm, pos_v, idx_v, buf_v, sem):
    wid = lax.axis_index("s") * NUM_CORES + lax.axis_index("c")
    pltpu.sync_copy(pos_hbm, pos_v)
    base_seq = wid * SEQ_PER_WORKER

    def seq_body(i, carry):
        row0 = (base_seq + i) * MAX_SEQ
        pltpu.sync_copy(idx_hbm.at[pl.ds(row0, MAX_SEQ)], idx_v)
        cp0 = pltpu.async_copy(
            tok_hbm.at[idx_v.at[pl.ds(0, 128)]], buf_v.at[pl.ds(0, 128)], sem
        )
        cp1 = pltpu.async_copy(
            tok_hbm.at[idx_v.at[pl.ds(128, 72)]], buf_v.at[pl.ds(128, 72)], sem
        )
        cp0.wait()
        cp1.wait()

        def add_body(r, c2):
            for c in range(VREGS_PER_ROW):
                sl = pl.ds(c * LANES, LANES)
                buf_v[r, sl] = buf_v[r, sl] + pos_v[r, sl]
            return c2

        lax.fori_loop(0, MAX_SEQ, add_body, 0, unroll=2)
        pltpu.sync_copy(buf_v, out_hbm.at[pl.ds(row0, MAX_SEQ)])
        return carry

    lax.fori_loop(0, SEQ_PER_WORKER, seq_body, 0)


def kernel(x, token_table, pos_table):
    idx_flat = x.reshape(BATCH * MAX_SEQ).astype(jnp.int32)
    out = _embed(idx_flat, token_table, pos_table)
    return out.reshape(BATCH, MAX_SEQ, EMBED)


# speedup vs baseline: 1.4262x; 1.4262x over previous
import functools

import jax
import jax.numpy as jnp
from jax import lax
from jax.experimental import pallas as pl
from jax.experimental.pallas import tpu as pltpu
from jax.experimental.pallas import tpu_sc as plsc

VOCAB = 1000000
EMBED = 64
MAX_SEQ = 200
BATCH = 4096

NUM_CORES = 2
NUM_SUBCORES = 16
NUM_WORKERS = NUM_CORES * NUM_SUBCORES
SEQ_PER_WORKER = BATCH // NUM_WORKERS
LANES = 16
VREGS_PER_ROW = EMBED // LANES

SEQ_PER_CHUNK = 4
ROWS = SEQ_PER_CHUNK * MAX_SEQ
NCHUNK = SEQ_PER_WORKER // SEQ_PER_CHUNK
SUB = 80
NSUB = ROWS // SUB

_mesh = plsc.VectorSubcoreMesh(core_axis_name="c", subcore_axis_name="s")


@functools.partial(
    pl.kernel,
    mesh=_mesh,
    compiler_params=pltpu.CompilerParams(use_tc_tiling_on_sc=False),
    out_type=jax.ShapeDtypeStruct((BATCH * MAX_SEQ, EMBED), jnp.float32),
    scratch_types=[
        pltpu.VMEM((MAX_SEQ, EMBED), jnp.float32),
        pltpu.VMEM((ROWS,), jnp.int32),
        pltpu.VMEM((ROWS,), jnp.int32),
        pltpu.VMEM((ROWS, EMBED), jnp.float32),
        pltpu.VMEM((ROWS, EMBED), jnp.float32),
        pltpu.SemaphoreType.DMA,
        pltpu.SemaphoreType.DMA,
        pltpu.SemaphoreType.DMA,
        pltpu.SemaphoreType.DMA,
    ],
)
def _embed(idx_hbm, tok_hbm, pos_hbm, out_hbm,
           pos_v, idx_a, idx_b, buf_a, buf_b,
           g_sem_a, g_sem_b, s_sem_a, s_sem_b):
    wid = lax.axis_index("s") * NUM_CORES + lax.axis_index("c")
    pltpu.sync_copy(pos_hbm, pos_v)
    base_row = wid * (SEQ_PER_WORKER * MAX_SEQ)

    def fire_chunk(c, idx_v, buf_v, g_sem):
        row0 = base_row + c * ROWS
        pltpu.sync_copy(idx_hbm.at[pl.ds(row0, ROWS)], idx_v)
        for j in range(NSUB):
            sl = pl.ds(j * SUB, SUB)
            pltpu.async_copy(tok_hbm.at[idx_v.at[sl]], buf_v.at[sl], g_sem)

    def drain_gather(idx_v, buf_v, g_sem):
        for j in range(NSUB):
            sl = pl.ds(j * SUB, SUB)
            pltpu.make_async_copy(
                tok_hbm.at[idx_v.at[sl]], buf_v.at[sl], g_sem
            ).wait()

    def add_pos(buf_v):
        def add_body(r, carry):
            for q in range(SEQ_PER_CHUNK):
                for c in range(VREGS_PER_ROW):
                    sl = pl.ds(c * LANES, LANES)
                    plsc.addupdate(buf_v.at[q * MAX_SEQ + r, sl], pos_v[r, sl])
            return carry

        lax.fori_loop(0, MAX_SEQ, add_body, 0, unroll=2)

    def fire_store(c, buf_v, s_sem):
        row0 = base_row + c * ROWS
        pltpu.async_copy(buf_v, out_hbm.at[pl.ds(row0, ROWS)], s_sem)

    def wait_store(buf_v, s_sem):
        pltpu.make_async_copy(buf_v, out_hbm.at[pl.ds(0, ROWS)], s_sem).wait()

    fire_chunk(0, idx_a, buf_a, g_sem_a)
    fire_chunk(1, idx_b, buf_b, g_sem_b)
    drain_gather(idx_a, buf_a, g_sem_a)
    add_pos(buf_a)
    fire_store(0, buf_a, s_sem_a)
    wait_store(buf_a, s_sem_a)
    fire_chunk(2, idx_a, buf_a, g_sem_a)
    drain_gather(idx_b, buf_b, g_sem_b)
    add_pos(buf_b)
    fire_store(1, buf_b, s_sem_b)

    npairs = NCHUNK // 2

    def pair_body(p, carry):
        e = 2 * p
        o = e + 1
        wait_store(buf_b, s_sem_b)
        fire_chunk(o, idx_b, buf_b, g_sem_b)
        drain_gather(idx_a, buf_a, g_sem_a)
        add_pos(buf_a)
        fire_store(e, buf_a, s_sem_a)

        @pl.when(p < npairs - 1)
        def _():
            wait_store(buf_a, s_sem_a)
            fire_chunk(e + 2, idx_a, buf_a, g_sem_a)

        drain_gather(idx_b, buf_b, g_sem_b)
        add_pos(buf_b)
        fire_store(o, buf_b, s_sem_b)
        return carry

    lax.fori_loop(1, npairs, pair_body, 0)
    wait_store(buf_a, s_sem_a)
    wait_store(buf_b, s_sem_b)


def kernel(x, token_table, pos_table):
    idx_flat = x.reshape(BATCH * MAX_SEQ).astype(jnp.int32)
    out = _embed(idx_flat, token_table, pos_table)
    return out.reshape(BATCH, MAX_SEQ, EMBED)
